# baseline (speedup 1.0000x reference)
import functools

import jax
import jax.numpy as jnp
from jax import lax
from jax.experimental import pallas as pl
from jax.experimental.pallas import tpu as pltpu
from jax.experimental.pallas import tpu_sc as plsc

BATCH = 16384
D = 32
V = 1_000_000
NC = 2
NS = 16
L = 16
NW = NC * NS

SC_N = 10240
BPW = SC_N // NW
K = 8
NCHK = BPW // K

TC_N = BATCH - SC_N
TCK = 128

_mesh = plsc.VectorSubcoreMesh(core_axis_name="c", subcore_axis_name="s")


@functools.partial(
    pl.kernel,
    mesh=_mesh,
    compiler_params=pltpu.CompilerParams(needs_layout_passes=False),
    out_type=jax.ShapeDtypeStruct((SC_N,), jnp.float32),
    scratch_types=[
        pltpu.VMEM((BPW,), jnp.int32),
        pltpu.VMEM((BPW,), jnp.int32),
        pltpu.VMEM((2 * K * D, 128), jnp.float32),
        pltpu.VMEM((D, BPW), jnp.float32),
        pltpu.VMEM((D, BPW), jnp.float32),
        pltpu.VMEM((BPW,), jnp.float32),
        pltpu.SemaphoreType.DMA,
    ],
)
def _ncf_sc(uidx_hbm, iidx_hbm, utab_hbm, itab_hbm, out_hbm,
            uidx_v, iidx_v, dst_v, ucomp_v, icomp_v, scores_v, sem):
    wid = lax.axis_index("s") * NC + lax.axis_index("c")
    base = wid * BPW

    pltpu.sync_copy(uidx_hbm.at[pl.ds(base, BPW)], uidx_v)
    pltpu.sync_copy(iidx_hbm.at[pl.ds(base, BPW)], iidx_v)

    lane = lax.iota(jnp.int32, L)

    def make_phase(idx_v, tab_hbm, comp_v):
        def fire(ci, slot):
            vec = idx_v[pl.ds((ci // 2) * (2 * K), 2 * K)]
            for k in range(K):
                ii = jnp.sum(jnp.where(lane == lax.rem(ci, 2) * K + k, vec, 0))
                su = pl.multiple_of((ii >> 7) * 128, 128)
                pltpu.async_copy(
                    tab_hbm.at[:, pl.ds(su, 128)],
                    dst_v.at[pl.ds(slot * (K * D) + k * D, D), :], sem)

        def chunk(c, carry):
            par = lax.rem(c, 2)
            nxt = lax.rem(c + 1, 2)

            @pl.when(c == 0)
            def _prime():
                fire(0, 0)

            for k in range(K):
                pltpu.make_async_copy(
                    tab_hbm.at[:, pl.ds(0, 128)],
                    dst_v.at[pl.ds(par * (K * D) + k * D, D), :], sem).wait()

            @pl.when(c + 1 < NCHK)
            def _next():
                fire(c + 1, nxt)

            vec = idx_v[pl.ds((c // 2) * (2 * K), 2 * K)]
            for k in range(K):
                ii = jnp.sum(jnp.where(lane == par * K + k, vec, 0))
                lu = ii & 127
                p = c * K + k
                for h in range(2):
                    rows = par * (K * D) + k * D + h * L + lane
                    vals = plsc.load_gather(
                        dst_v, [rows, jnp.full((L,), 0, jnp.int32) + lu])
                    plsc.store_scatter(
                        comp_v, [h * L + lane, jnp.full((L,), 0, jnp.int32) + p],
                        vals)
            return carry
        return chunk

    lax.fori_loop(0, NCHK, make_phase(uidx_v, utab_hbm, ucomp_v), 0)
    lax.fori_loop(0, NCHK, make_phase(iidx_v, itab_hbm, icomp_v), 0)

    def dot(p, carry):
        acc = jnp.zeros((L,), jnp.float32)
        for j in range(D):
            acc = acc + (ucomp_v[j, pl.ds(p * L, L)]
                         * icomp_v[j, pl.ds(p * L, L)])
        scores_v[pl.ds(p * L, L)] = acc
        return carry

    lax.fori_loop(0, BPW // L, dot, 0)

    pltpu.sync_copy(scores_v, out_hbm.at[pl.ds(base, BPW)])


def _tc_body(uidx_s, iidx_s, uidx_b, iidx_b, utab, itab, out_b,
             uwin, iwin, sem_u, sem_i):
    step = pl.program_id(0)
    nstep = TC_N // TCK
    par = lax.rem(step, 2)
    nxt = lax.rem(step + 1, 2)

    def fire(ci, slot):
        for k in range(TCK):
            iu = uidx_s[SC_N + ci * TCK + k]
            ii = iidx_s[SC_N + ci * TCK + k]
            su = pl.multiple_of((iu >> 7) * 128, 128)
            si = pl.multiple_of((ii >> 7) * 128, 128)
            pltpu.make_async_copy(
                utab.at[:, pl.ds(su, 128)], uwin.at[slot, k], sem_u).start()
            pltpu.make_async_copy(
                itab.at[:, pl.ds(si, 128)], iwin.at[slot, k], sem_i).start()

    @pl.when(step == 0)
    def _prime():
        fire(0, 0)

    for k in range(TCK):
        pltpu.make_async_copy(
            utab.at[:, pl.ds(0, 128)], uwin.at[par, k], sem_u).wait()
        pltpu.make_async_copy(
            itab.at[:, pl.ds(0, 128)], iwin.at[par, k], sem_i).wait()

    @pl.when(step + 1 < nstep)
    def _next():
        fire(step + 1, nxt)

    lu = uidx_b[...] & 127
    li = iidx_b[...] & 127
    lanes = lax.broadcasted_iota(jnp.int32, (TCK, 1, 128), 2)
    um = (lanes == lu[:, None, None]).astype(jnp.float32)
    im = (lanes == li[:, None, None]).astype(jnp.float32)
    ones = jnp.ones((128, 8), jnp.float32)
    uvals = jnp.dot((uwin[par] * um).reshape(TCK * D, 128), ones,
                    preferred_element_type=jnp.float32)[:, 0].reshape(TCK, D)
    ivals = jnp.dot((iwin[par] * im).reshape(TCK * D, 128), ones,
                    preferred_element_type=jnp.float32)[:, 0].reshape(TCK, D)
    out_b[...] = jnp.sum(uvals * ivals, axis=1)


_tc_call = pl.pallas_call(
    _tc_body,
    grid=(TC_N // TCK,),
    in_specs=[
        pl.BlockSpec(memory_space=pltpu.SMEM),
        pl.BlockSpec(memory_space=pltpu.SMEM),
        pl.BlockSpec((TCK,), lambda i: (SC_N // TCK + i,)),
        pl.BlockSpec((TCK,), lambda i: (SC_N // TCK + i,)),
        pl.BlockSpec(memory_space=pl.ANY),
        pl.BlockSpec(memory_space=pl.ANY),
    ],
    out_specs=pl.BlockSpec((TCK,), lambda i: (i,)),
    out_shape=jax.ShapeDtypeStruct((TC_N,), jnp.float32),
    scratch_shapes=[
        pltpu.VMEM((2, TCK, D, 128), jnp.float32),
        pltpu.VMEM((2, TCK, D, 128), jnp.float32),
        pltpu.SemaphoreType.DMA,
        pltpu.SemaphoreType.DMA,
    ],
)


def kernel(user_idx, item_idx, user_table, item_table):
    u32 = user_idx.astype(jnp.int32)
    i32 = item_idx.astype(jnp.int32)
    ut = user_table.T
    it = item_table.T
    sc_scores = _ncf_sc(u32, i32, ut, it)
    tc_scores = _tc_call(u32, i32, u32, i32, ut, it)
    return jnp.concatenate([sc_scores, tc_scores])


# speedup vs baseline: 1.0980x; 1.0980x over previous
import functools

import jax
import jax.numpy as jnp
from jax import lax
from jax.experimental import pallas as pl
from jax.experimental.pallas import tpu as pltpu
from jax.experimental.pallas import tpu_sc as plsc

BATCH = 16384
D = 32
V = 1_000_000
NC = 2
NS = 16
L = 16
NW = NC * NS

SC_N = 9728
BPW = SC_N // NW
K = 16
NCHK = BPW // K

TC_N = BATCH - SC_N
TCK = 128

_mesh = plsc.VectorSubcoreMesh(core_axis_name="c", subcore_axis_name="s")


@functools.partial(
    pl.kernel,
    mesh=_mesh,
    compiler_params=pltpu.CompilerParams(needs_layout_passes=False),
    out_type=jax.ShapeDtypeStruct((SC_N,), jnp.float32),
    scratch_types=[
        pltpu.VMEM((BPW,), jnp.int32),
        pltpu.VMEM((BPW,), jnp.int32),
        pltpu.VMEM((K * D, 128), jnp.float32),
        pltpu.VMEM((D, BPW), jnp.float32),
        pltpu.VMEM((D, BPW), jnp.float32),
        pltpu.VMEM((BPW,), jnp.float32),
        pltpu.SemaphoreType.DMA,
    ],
)
def _ncf_sc(uidx_hbm, iidx_hbm, utab_hbm, itab_hbm, out_hbm,
            uidx_v, iidx_v, dst_v, ucomp_v, icomp_v, scores_v, sem):
    wid = lax.axis_index("s") * NC + lax.axis_index("c")
    base = wid * BPW

    pltpu.sync_copy(uidx_hbm.at[pl.ds(base, BPW)], uidx_v)
    pltpu.sync_copy(iidx_hbm.at[pl.ds(base, BPW)], iidx_v)

    lane = lax.iota(jnp.int32, L)

    def make_phase(idx_v, tab_hbm, comp_v):
        def chunk(c, carry):
            vec = idx_v[pl.ds(c * K, K)]
            cps = []
            for k in range(K):
                ii = jnp.sum(jnp.where(lane == k, vec, 0))
                su = pl.multiple_of((ii >> 7) * 128, 128)
                cps.append(pltpu.async_copy(
                    tab_hbm.at[:, pl.ds(su, 128)],
                    dst_v.at[pl.ds(k * D, D), :], sem))
            for cp in cps:
                cp.wait()
            for k in range(K):
                ii = jnp.sum(jnp.where(lane == k, vec, 0))
                lu = ii & 127
                p = c * K + k
                for h in range(2):
                    rows = k * D + h * L + lane
                    vals = plsc.load_gather(
                        dst_v, [rows, jnp.full((L,), 0, jnp.int32) + lu])
                    plsc.store_scatter(
                        comp_v, [h * L + lane, jnp.full((L,), 0, jnp.int32) + p],
                        vals)
            return carry
        return chunk

    lax.fori_loop(0, NCHK, make_phase(uidx_v, utab_hbm, ucomp_v), 0)
    lax.fori_loop(0, NCHK, make_phase(iidx_v, itab_hbm, icomp_v), 0)

    def dot(p, carry):
        acc = jnp.zeros((L,), jnp.float32)
        for j in range(D):
            acc = acc + (ucomp_v[j, pl.ds(p * L, L)]
                         * icomp_v[j, pl.ds(p * L, L)])
        scores_v[pl.ds(p * L, L)] = acc
        return carry

    lax.fori_loop(0, BPW // L, dot, 0)

    pltpu.sync_copy(scores_v, out_hbm.at[pl.ds(base, BPW)])


def _tc_body(uidx_s, iidx_s, uidx_b, iidx_b, utab, itab, out_b,
             uwin, iwin, sem_u, sem_i):
    step = pl.program_id(0)
    nstep = TC_N // TCK
    par = lax.rem(step, 2)
    nxt = lax.rem(step + 1, 2)

    def fire(ci, slot):
        for k in range(TCK):
            iu = uidx_s[SC_N + ci * TCK + k]
            ii = iidx_s[SC_N + ci * TCK + k]
            su = pl.multiple_of((iu >> 7) * 128, 128)
            si = pl.multiple_of((ii >> 7) * 128, 128)
            pltpu.make_async_copy(
                utab.at[:, pl.ds(su, 128)], uwin.at[slot, k], sem_u).start()
            pltpu.make_async_copy(
                itab.at[:, pl.ds(si, 128)], iwin.at[slot, k], sem_i).start()

    @pl.when(step == 0)
    def _prime():
        fire(0, 0)

    for k in range(TCK):
        pltpu.make_async_copy(
            utab.at[:, pl.ds(0, 128)], uwin.at[par, k], sem_u).wait()
        pltpu.make_async_copy(
            itab.at[:, pl.ds(0, 128)], iwin.at[par, k], sem_i).wait()

    @pl.when(step + 1 < nstep)
    def _next():
        fire(step + 1, nxt)

    lu = uidx_b[...] & 127
    li = iidx_b[...] & 127
    lanes = lax.broadcasted_iota(jnp.int32, (TCK, 1, 128), 2)
    um = (lanes == lu[:, None, None]).astype(jnp.float32)
    im = (lanes == li[:, None, None]).astype(jnp.float32)
    ones = jnp.ones((128, 8), jnp.float32)
    uvals = jnp.dot((uwin[par] * um).reshape(TCK * D, 128), ones,
                    preferred_element_type=jnp.float32)[:, 0].reshape(TCK, D)
    ivals = jnp.dot((iwin[par] * im).reshape(TCK * D, 128), ones,
                    preferred_element_type=jnp.float32)[:, 0].reshape(TCK, D)
    out_b[...] = jnp.sum(uvals * ivals, axis=1)


_tc_call = pl.pallas_call(
    _tc_body,
    grid=(TC_N // TCK,),
    in_specs=[
        pl.BlockSpec(memory_space=pltpu.SMEM),
        pl.BlockSpec(memory_space=pltpu.SMEM),
        pl.BlockSpec((TCK,), lambda i: (SC_N // TCK + i,)),
        pl.BlockSpec((TCK,), lambda i: (SC_N // TCK + i,)),
        pl.BlockSpec(memory_space=pl.ANY),
        pl.BlockSpec(memory_space=pl.ANY),
    ],
    out_specs=pl.BlockSpec((TCK,), lambda i: (i,)),
    out_shape=jax.ShapeDtypeStruct((TC_N,), jnp.float32),
    scratch_shapes=[
        pltpu.VMEM((2, TCK, D, 128), jnp.float32),
        pltpu.VMEM((2, TCK, D, 128), jnp.float32),
        pltpu.SemaphoreType.DMA,
        pltpu.SemaphoreType.DMA,
    ],
)


def kernel(user_idx, item_idx, user_table, item_table):
    u32 = user_idx.astype(jnp.int32)
    i32 = item_idx.astype(jnp.int32)
    ut = user_table.T
    it = item_table.T
    sc_scores = _ncf_sc(u32, i32, ut, it)
    tc_scores = _tc_call(u32, i32, u32, i32, ut, it)
    return jnp.concatenate([sc_scores, tc_scores])
